# baseline (speedup 1.0000x reference)
import jax
import jax.numpy as jnp
from jax.experimental import pallas as pl
from jax.experimental.pallas import tpu as pltpu

_TOP_K = 8
_N_EXPERTS = 64
_DIM = 768


def _topk_from_logits(logits):
    vals = logits.T
    iota = jax.lax.broadcasted_iota(jnp.int32, vals.shape, 0)
    m = None
    top_vals = []
    top_idx = []
    for _ in range(_TOP_K):
        mk = jnp.max(vals, axis=0, keepdims=True)
        ik = jnp.min(
            jnp.where(vals == mk, iota, _N_EXPERTS), axis=0, keepdims=True
        )
        if m is None:
            m = mk
        top_vals.append(jnp.exp(mk - m))
        top_idx.append(ik)
        vals = jnp.where(iota == ik, -jnp.inf, vals)
    w = jnp.concatenate(top_vals, axis=0)
    i = jnp.concatenate(top_idx, axis=0)
    w = w / jnp.sum(w, axis=0, keepdims=True)
    return i.T, w.T


_CHUNK = 1024
_NBUF = 4


def _gate_kernel(x_hbm, wt_ref, idx_ref, w_ref, buf, sem):
    n_chunks = x_hbm.shape[0] // _CHUNK
    wt = wt_ref[...]

    def copy_in(c, slot):
        return pltpu.make_async_copy(
            x_hbm.at[pl.ds(c * _CHUNK, _CHUNK), :],
            buf.at[slot],
            sem.at[slot],
        )

    for c in range(_NBUF):
        copy_in(c, c).start()

    def body(c, carry):
        slot = jax.lax.rem(c, _NBUF)
        copy_in(c, slot).wait()
        x = buf[slot]
        logits = jnp.dot(x, wt, preferred_element_type=jnp.float32)
        nc = c + _NBUF

        @pl.when(nc < n_chunks)
        def _():
            copy_in(nc, slot).start()

        i, w = _topk_from_logits(logits)
        idx_ref[pl.ds(c * _CHUNK, _CHUNK), :] = i
        w_ref[pl.ds(c * _CHUNK, _CHUNK), :] = w
        return carry

    jax.lax.fori_loop(0, n_chunks, body, 0)


@jax.jit
def _gate(x, wt):
    n_tokens = x.shape[0]
    idx, w = pl.pallas_call(
        _gate_kernel,
        in_specs=[
            pl.BlockSpec(memory_space=pl.ANY),
            pl.BlockSpec(memory_space=pltpu.VMEM),
        ],
        out_specs=[
            pl.BlockSpec(memory_space=pltpu.VMEM),
            pl.BlockSpec(memory_space=pltpu.VMEM),
        ],
        out_shape=[
            jax.ShapeDtypeStruct((n_tokens, _TOP_K), jnp.int32),
            jax.ShapeDtypeStruct((n_tokens, _TOP_K), jnp.float32),
        ],
        scratch_shapes=[
            pltpu.VMEM((_NBUF, _CHUNK, _DIM), jnp.float32),
            pltpu.SemaphoreType.DMA((_NBUF,)),
        ],
    )(x, wt)
    return idx, w


def kernel(hidden_states, weight):
    bsz, seq_len, h = hidden_states.shape
    x = hidden_states.reshape(-1, h)
    idx, w = _gate(x, weight.T)
    return idx, w, jnp.float32(0.0)


# speedup vs baseline: 1.0684x; 1.0684x over previous
import jax
import jax.numpy as jnp
from jax.experimental import pallas as pl
from jax.experimental.pallas import tpu as pltpu

_TOP_K = 8
_N_EXPERTS = 64
_DIM = 768


def _topk_from_logits(logits):
    vals = logits.T
    iota = jax.lax.broadcasted_iota(jnp.int32, vals.shape, 0)
    m = None
    top_vals = []
    top_idx = []
    for _ in range(_TOP_K):
        mk = jnp.max(vals, axis=0, keepdims=True)
        ik = jnp.min(
            jnp.where(vals == mk, iota, _N_EXPERTS), axis=0, keepdims=True
        )
        if m is None:
            m = mk
        top_vals.append(jnp.exp(mk - m))
        top_idx.append(ik)
        vals = jnp.where(iota == ik, -jnp.inf, vals)
    w = jnp.concatenate(top_vals, axis=0)
    i = jnp.concatenate(top_idx, axis=0)
    w = w / jnp.sum(w, axis=0, keepdims=True)
    return i.T, w.T


_CHUNK = 2048
_NBUF = 4


def _gate_kernel(x_hbm, wt_ref, idx_ref, w_ref, buf, sem):
    n_chunks = x_hbm.shape[0] // _CHUNK
    wt = wt_ref[...]

    def copy_in(c, slot):
        return pltpu.make_async_copy(
            x_hbm.at[pl.ds(c * _CHUNK, _CHUNK), :],
            buf.at[slot],
            sem.at[slot],
        )

    for c in range(_NBUF):
        copy_in(c, c).start()

    def body(c, carry):
        slot = jax.lax.rem(c, _NBUF)
        copy_in(c, slot).wait()
        x = buf[slot]
        logits = jnp.dot(x, wt, preferred_element_type=jnp.float32)
        nc = c + _NBUF

        @pl.when(nc < n_chunks)
        def _():
            copy_in(nc, slot).start()

        i, w = _topk_from_logits(logits)
        idx_ref[pl.ds(c * _CHUNK, _CHUNK), :] = i
        w_ref[pl.ds(c * _CHUNK, _CHUNK), :] = w
        return carry

    jax.lax.fori_loop(0, n_chunks, body, 0)


@jax.jit
def _gate(x, wt):
    n_tokens = x.shape[0]
    idx, w = pl.pallas_call(
        _gate_kernel,
        in_specs=[
            pl.BlockSpec(memory_space=pl.ANY),
            pl.BlockSpec(memory_space=pltpu.VMEM),
        ],
        out_specs=[
            pl.BlockSpec(memory_space=pltpu.VMEM),
            pl.BlockSpec(memory_space=pltpu.VMEM),
        ],
        out_shape=[
            jax.ShapeDtypeStruct((n_tokens, _TOP_K), jnp.int32),
            jax.ShapeDtypeStruct((n_tokens, _TOP_K), jnp.float32),
        ],
        scratch_shapes=[
            pltpu.VMEM((_NBUF, _CHUNK, _DIM), jnp.float32),
            pltpu.SemaphoreType.DMA((_NBUF,)),
        ],
    )(x, wt)
    return idx, w


def kernel(hidden_states, weight):
    bsz, seq_len, h = hidden_states.shape
    x = hidden_states.reshape(-1, h)
    idx, w = _gate(x, weight.T)
    return idx, w, jnp.float32(0.0)


# speedup vs baseline: 1.1186x; 1.0470x over previous
import jax
import jax.numpy as jnp
from jax.experimental import pallas as pl
from jax.experimental.pallas import tpu as pltpu

_TOP_K = 8
_N_EXPERTS = 64
_DIM = 768


def _topk_from_logits(logits):
    vals = logits.T
    iota = jax.lax.broadcasted_iota(jnp.int32, vals.shape, 0)
    m = None
    top_vals = []
    top_idx = []
    for _ in range(_TOP_K):
        mk = jnp.max(vals, axis=0, keepdims=True)
        ik = jnp.min(
            jnp.where(vals == mk, iota, _N_EXPERTS), axis=0, keepdims=True
        )
        if m is None:
            m = mk
        top_vals.append(jnp.exp(mk - m))
        top_idx.append(ik)
        vals = jnp.where(iota == ik, -jnp.inf, vals)
    w = jnp.concatenate(top_vals, axis=0)
    i = jnp.concatenate(top_idx, axis=0)
    w = w / jnp.sum(w, axis=0, keepdims=True)
    return i.T, w.T


_SUB = 512


def _gate_kernel(x_ref, wt_ref, idx_ref, w_ref):
    logits = jnp.dot(
        x_ref[...], wt_ref[...], preferred_element_type=jnp.float32
    )
    tb = logits.shape[0]
    for s in range(tb // _SUB):
        i, w = _topk_from_logits(logits[s * _SUB : (s + 1) * _SUB])
        idx_ref[s * _SUB : (s + 1) * _SUB, :] = i
        w_ref[s * _SUB : (s + 1) * _SUB, :] = w


@jax.jit
def _gate(x, wt):
    n_tokens = x.shape[0]
    tb = 4096
    grid = (n_tokens // tb,)
    idx, w = pl.pallas_call(
        _gate_kernel,
        grid=grid,
        in_specs=[
            pl.BlockSpec((tb, _DIM), lambda i: (i, 0)),
            pl.BlockSpec((_DIM, _N_EXPERTS), lambda i: (0, 0)),
        ],
        out_specs=[
            pl.BlockSpec((tb, _TOP_K), lambda i: (i, 0)),
            pl.BlockSpec((tb, _TOP_K), lambda i: (i, 0)),
        ],
        out_shape=[
            jax.ShapeDtypeStruct((n_tokens, _TOP_K), jnp.int32),
            jax.ShapeDtypeStruct((n_tokens, _TOP_K), jnp.float32),
        ],
        compiler_params=pltpu.CompilerParams(
            dimension_semantics=("arbitrary",),
        ),
    )(x, wt)
    return idx, w


def kernel(hidden_states, weight):
    bsz, seq_len, h = hidden_states.shape
    x = hidden_states.reshape(-1, h)
    idx, w = _gate(x, weight.T)
    return idx, w, jnp.float32(0.0)
